# baseline (speedup 1.0000x reference)
import jax
import jax.numpy as jnp
from jax import lax
from jax.experimental import pallas as pl
from jax.experimental.pallas import tpu as pltpu
from jax.experimental.pallas import tpu_sc as plsc

_NUM_FIELDS = 26
_VOCAB = 100000
_VPAD = 100096
_EMB_DIM = 50
_EMB_PAD = 56
_BATCH = 16384

_NC = 2
_NS = 16
_NW = _NC * _NS

_VT = _VPAD // 128 - 1
_VTAIL0 = _VT * 128
_NTAIL = _VOCAB - _VTAIL0
_UNIT_W = 128 * _EMB_PAD
_TAIL_W = _NTAIL * _EMB_PAD
_SCRATCH_ROWS = _NUM_FIELDS * _VPAD
_SCRATCH_W = _SCRATCH_ROWS * _EMB_PAD

_ROWS = _BATCH * _NUM_FIELDS
_PER_W = _ROWS // _NW
_CH = 128
_K = _PER_W // _CH
_NBUF = 4


def _repack_body(tt_hbm, app_hbm, out_hbm, in_v, mid0, mid1, out_v, *sems):
    mids = (mid0, mid1)
    sis = sems[:2]
    sos = sems[2:]
    wid = lax.axis_index("s") * _NC + lax.axis_index("c")
    n_k = (_VT - wid + _NW - 1) // _NW
    n_total = _NUM_FIELDS * n_k

    u16 = lax.iota(jnp.int32, 16)
    iota57 = u16 * 57
    pat2 = []
    for p in range(7):
        up = u16 + (16 * p)
        pat2.append(up + up // _EMB_PAD)

    def _issue_in(f, vt, b):
        pltpu.async_copy(
            tt_hbm.at[f, :, pl.ds(pl.multiple_of(vt * 128, 128), 128)],
            in_v.at[b, pl.ds(0, _EMB_DIM)],
            sis[b],
        )

    def _advance(f, vt):
        nvt = vt + _NW
        wrap = nvt >= _VT
        return jnp.where(wrap, f + 1, f), jnp.where(wrap, jnp.int32(wid), nvt)

    _issue_in(jnp.int32(0), jnp.int32(wid), 0)
    f1, vt1 = _advance(jnp.int32(0), jnp.int32(wid))
    _issue_in(f1, vt1, 1)

    @pl.loop(0, n_total, step=2, init_carry=(jnp.int32(0), jnp.int32(wid)))
    def _(t0, carry):
        f, vt = carry
        for b in range(2):
            t = t0 + b
            pltpu.make_async_copy(
                tt_hbm.at[f, :, pl.ds(pl.multiple_of(vt * 128, 128), 128)],
                in_v.at[b, pl.ds(0, _EMB_DIM)],
                sis[b],
            ).wait()

            @pl.when(t >= 2)
            def _():
                pltpu.make_async_copy(
                    out_v.at[b], out_hbm.at[pl.ds(0, _UNIT_W)], sos[b]
                ).wait()

            @plsc.parallel_loop(0, 8)
            def _(vb):
                v0 = pl.multiple_of(vb * 16, 16)
                acc = iota57 + vb * (16 * 57)
                for d in range(_EMB_PAD):
                    x = in_v[b, d, pl.ds(v0, 16)]
                    plsc.store_scatter(mids[b], [acc], x)
                    if d + 1 < _EMB_PAD:
                        acc = acc + 1

            @plsc.parallel_loop(0, _UNIT_W // 112, unroll=8)
            def _(q):
                sq = 114 * q
                for p in range(7):
                    x = plsc.load_gather(mids[b], [pat2[p] + sq])
                    out_v[b, pl.ds(112 * q + 16 * p, 16)] = x

            off = (f * _VPAD + vt * 128) * _EMB_PAD
            pltpu.async_copy(
                out_v.at[b], out_hbm.at[pl.ds(off, _UNIT_W)], sos[b]
            )

            f, vt = _advance(f, vt)

            @pl.when(t + 2 < n_total)
            def _():
                f3, vt3 = _advance(f, vt)
                _issue_in(f3, vt3, b)

        return f, vt

    for b in range(2):
        pltpu.make_async_copy(
            out_v.at[b], out_hbm.at[pl.ds(0, _UNIT_W)], sos[b]
        ).wait()

    @pl.when(wid < _NUM_FIELDS)
    def _():
        pltpu.sync_copy(
            app_hbm.at[pl.ds(wid * _TAIL_W, _TAIL_W)],
            out_v.at[0, pl.ds(0, _TAIL_W)],
        )
        t_off = (wid * _VPAD + _VTAIL0) * _EMB_PAD
        pltpu.sync_copy(
            out_v.at[0, pl.ds(0, _TAIL_W)], out_hbm.at[pl.ds(t_off, _TAIL_W)]
        )


def _gather_body(tab_hbm, idx_hbm, out_hbm, idx_v, rows_v, *sems):
    wid = lax.axis_index("s") * _NC + lax.axis_index("c")
    ubase = wid * _K
    pltpu.sync_copy(idx_hbm.at[pl.ds(ubase, _K)], idx_v)
    for b in range(_NBUF):
        pltpu.async_copy(tab_hbm.at[idx_v.at[b]], rows_v.at[b], sems[b])

    @pl.loop(0, _K, step=_NBUF)
    def _(j0):
        for b in range(_NBUF):
            j = j0 + b
            u = ubase + j
            f = u // 128
            bb = u % 128
            pltpu.make_async_copy(
                tab_hbm.at[idx_v.at[b]], rows_v.at[b], sems[b]
            ).wait()
            pltpu.sync_copy(
                rows_v.at[b],
                out_hbm.at[pl.ds(bb * _CH, _CH), pl.ds(f * _EMB_PAD, _EMB_PAD)],
            )
            nj = j + _NBUF

            @pl.when(nj < _K)
            def _():
                pltpu.async_copy(tab_hbm.at[idx_v.at[nj]], rows_v.at[b], sems[b])


@jax.jit
def kernel(x_cat, tables):
    tt = jnp.transpose(tables, (0, 2, 1))

    app = jnp.pad(
        tables[:, _VTAIL0:, :], ((0, 0), (0, 0), (0, _EMB_PAD - _EMB_DIM))
    ).reshape(-1)

    mesh = plsc.VectorSubcoreMesh(core_axis_name="c", subcore_axis_name="s")
    scratch_flat = pl.kernel(
        _repack_body,
        out_type=jax.ShapeDtypeStruct((_SCRATCH_W,), jnp.float32),
        mesh=mesh,
        scratch_types=[
            pltpu.VMEM((2, _EMB_PAD, 128), jnp.float32),
            pltpu.VMEM((57 * 128,), jnp.float32),
            pltpu.VMEM((57 * 128,), jnp.float32),
            pltpu.VMEM((2, _UNIT_W), jnp.float32),
        ] + [pltpu.SemaphoreType.DMA] * 4,
        compiler_params=pltpu.CompilerParams(needs_layout_passes=False),
    )(tt, app)

    tab = scratch_flat.reshape(_SCRATCH_ROWS, _EMB_PAD)
    offs = (jnp.arange(_NUM_FIELDS, dtype=jnp.int32) * _VPAD)[:, None]
    gidx = (x_cat.T.astype(jnp.int32) + offs).reshape(
        _NUM_FIELDS * _BATCH // _CH, _CH
    )

    out = pl.kernel(
        _gather_body,
        out_type=jax.ShapeDtypeStruct(
            (_BATCH, _NUM_FIELDS * _EMB_PAD), jnp.float32
        ),
        mesh=mesh,
        scratch_types=[
            pltpu.VMEM((_K, _CH), jnp.int32),
            pltpu.VMEM((_NBUF, _CH, _EMB_PAD), jnp.float32),
        ] + [pltpu.SemaphoreType.DMA] * _NBUF,
        compiler_params=pltpu.CompilerParams(use_tc_tiling_on_sc=False),
    )(tab, gidx)
    out3 = out.reshape(_BATCH, _NUM_FIELDS, _EMB_PAD)
    return out3[:, :, :_EMB_DIM].reshape(_BATCH, _NUM_FIELDS * _EMB_DIM)


# speedup vs baseline: 1.1119x; 1.1119x over previous
import jax
import jax.numpy as jnp
from jax import lax
from jax.experimental import pallas as pl
from jax.experimental.pallas import tpu as pltpu
from jax.experimental.pallas import tpu_sc as plsc

_NUM_FIELDS = 26
_VOCAB = 100000
_VPAD = 100096
_EMB_DIM = 50
_EMB_PAD = 56
_BATCH = 16384

_NC = 2
_NS = 16
_NW = _NC * _NS

_VT = _VPAD // 128 - 1
_VTAIL0 = _VT * 128
_NTAIL = _VOCAB - _VTAIL0
_UNIT_W = 128 * _EMB_PAD
_TAIL_W = _NTAIL * _EMB_PAD
_SCRATCH_ROWS = _NUM_FIELDS * _VPAD
_SCRATCH_W = _SCRATCH_ROWS * _EMB_PAD

_ROWS = _BATCH * _NUM_FIELDS
_PER_W = _ROWS // _NW
_CH = 128
_K = _PER_W // _CH
_NBUF = 4


def _repack_body(tt_hbm, app_hbm, out_hbm, in_v, mid0, mid1, out_v, *sems):
    mids = (mid0, mid1)
    sis = sems[:2]
    sos = sems[2:]
    wid = lax.axis_index("s") * _NC + lax.axis_index("c")
    n_k = (_VT - wid + _NW - 1) // _NW
    n_total = _NUM_FIELDS * n_k

    u16 = lax.iota(jnp.int32, 16)
    iota57 = u16 * 57
    pat2 = []
    for p in range(7):
        up = u16 + (16 * p)
        pat2.append(up + up // _EMB_PAD)

    def _issue_in(f, vt, b):
        pltpu.async_copy(
            tt_hbm.at[f, :, pl.ds(pl.multiple_of(vt * 128, 128), 128)],
            in_v.at[b, pl.ds(0, _EMB_DIM)],
            sis[b],
        )

    def _advance(f, vt):
        nvt = vt + _NW
        wrap = nvt >= _VT
        return jnp.where(wrap, f + 1, f), jnp.where(wrap, jnp.int32(wid), nvt)

    _issue_in(jnp.int32(0), jnp.int32(wid), 0)
    f1, vt1 = _advance(jnp.int32(0), jnp.int32(wid))
    _issue_in(f1, vt1, 1)

    @pl.loop(0, n_total, step=2, init_carry=(jnp.int32(0), jnp.int32(wid)))
    def _(t0, carry):
        f, vt = carry
        for b in range(2):
            t = t0 + b
            pltpu.make_async_copy(
                tt_hbm.at[f, :, pl.ds(pl.multiple_of(vt * 128, 128), 128)],
                in_v.at[b, pl.ds(0, _EMB_DIM)],
                sis[b],
            ).wait()

            @pl.when(t >= 2)
            def _():
                pltpu.make_async_copy(
                    out_v.at[b], out_hbm.at[pl.ds(0, _UNIT_W)], sos[b]
                ).wait()

            @plsc.parallel_loop(0, 8)
            def _(vb):
                base = vb * (16 * 57)
                v0 = pl.multiple_of(vb * 16, 16)
                for d in range(_EMB_PAD):
                    x = in_v[b, d, pl.ds(v0, 16)]
                    plsc.store_scatter(mids[b], [iota57 + (base + d)], x)

            @plsc.parallel_loop(0, _UNIT_W // 112, unroll=8)
            def _(q):
                sq = 114 * q
                for p in range(7):
                    x = plsc.load_gather(mids[b], [pat2[p] + sq])
                    out_v[b, pl.ds(112 * q + 16 * p, 16)] = x

            off = (f * _VPAD + vt * 128) * _EMB_PAD
            pltpu.async_copy(
                out_v.at[b], out_hbm.at[pl.ds(off, _UNIT_W)], sos[b]
            )

            f, vt = _advance(f, vt)

            @pl.when(t + 2 < n_total)
            def _():
                f3, vt3 = _advance(f, vt)
                _issue_in(f3, vt3, b)

        return f, vt

    for b in range(2):
        pltpu.make_async_copy(
            out_v.at[b], out_hbm.at[pl.ds(0, _UNIT_W)], sos[b]
        ).wait()

    @pl.when(wid < _NUM_FIELDS)
    def _():
        pltpu.sync_copy(
            app_hbm.at[pl.ds(wid * _TAIL_W, _TAIL_W)],
            out_v.at[0, pl.ds(0, _TAIL_W)],
        )
        t_off = (wid * _VPAD + _VTAIL0) * _EMB_PAD
        pltpu.sync_copy(
            out_v.at[0, pl.ds(0, _TAIL_W)], out_hbm.at[pl.ds(t_off, _TAIL_W)]
        )


def _gather_body(tab_hbm, idx_hbm, out_hbm, idx_v, rows_v, *sems):
    wid = lax.axis_index("s") * _NC + lax.axis_index("c")
    ubase = wid * _K
    pltpu.sync_copy(idx_hbm.at[pl.ds(ubase, _K)], idx_v)
    for b in range(_NBUF):
        pltpu.async_copy(tab_hbm.at[idx_v.at[b]], rows_v.at[b], sems[b])

    @pl.loop(0, _K, step=_NBUF)
    def _(j0):
        for b in range(_NBUF):
            j = j0 + b
            u = ubase + j
            f = u // 128
            bb = u % 128
            pltpu.make_async_copy(
                tab_hbm.at[idx_v.at[b]], rows_v.at[b], sems[b]
            ).wait()
            pltpu.sync_copy(
                rows_v.at[b],
                out_hbm.at[pl.ds(bb * _CH, _CH), pl.ds(f * _EMB_PAD, _EMB_PAD)],
            )
            nj = j + _NBUF

            @pl.when(nj < _K)
            def _():
                pltpu.async_copy(tab_hbm.at[idx_v.at[nj]], rows_v.at[b], sems[b])


@jax.jit
def kernel(x_cat, tables):
    tt = jnp.transpose(tables, (0, 2, 1))

    app = jnp.pad(
        tables[:, _VTAIL0:, :], ((0, 0), (0, 0), (0, _EMB_PAD - _EMB_DIM))
    ).reshape(-1)

    mesh = plsc.VectorSubcoreMesh(core_axis_name="c", subcore_axis_name="s")
    scratch_flat = pl.kernel(
        _repack_body,
        out_type=jax.ShapeDtypeStruct((_SCRATCH_W,), jnp.float32),
        mesh=mesh,
        scratch_types=[
            pltpu.VMEM((2, _EMB_PAD, 128), jnp.float32),
            pltpu.VMEM((57 * 128,), jnp.float32),
            pltpu.VMEM((57 * 128,), jnp.float32),
            pltpu.VMEM((2, _UNIT_W), jnp.float32),
        ] + [pltpu.SemaphoreType.DMA] * 4,
        compiler_params=pltpu.CompilerParams(needs_layout_passes=False),
    )(tt, app)

    tab = scratch_flat.reshape(_SCRATCH_ROWS, _EMB_PAD)
    offs = (jnp.arange(_NUM_FIELDS, dtype=jnp.int32) * _VPAD)[:, None]
    gidx = (x_cat.T.astype(jnp.int32) + offs).reshape(
        _NUM_FIELDS * _BATCH // _CH, _CH
    )

    out = pl.kernel(
        _gather_body,
        out_type=jax.ShapeDtypeStruct(
            (_BATCH, _NUM_FIELDS * _EMB_PAD), jnp.float32
        ),
        mesh=mesh,
        scratch_types=[
            pltpu.VMEM((_K, _CH), jnp.int32),
            pltpu.VMEM((_NBUF, _CH, _EMB_PAD), jnp.float32),
        ] + [pltpu.SemaphoreType.DMA] * _NBUF,
        compiler_params=pltpu.CompilerParams(use_tc_tiling_on_sc=False),
    )(tab, gidx)
    out3 = out.reshape(_BATCH, _NUM_FIELDS, _EMB_PAD)
    return out3[:, :, :_EMB_DIM].reshape(_BATCH, _NUM_FIELDS * _EMB_DIM)
